# baseline (speedup 1.0000x reference)
import functools
import jax
import jax.numpy as jnp
from jax import lax
from jax.experimental import pallas as pl
from jax.experimental.pallas import tpu as pltpu
from jax.experimental.pallas import tpu_sc as plsc

B = 4096
D0N, D1N = 10, 20
W1, W2, W3 = 32, 16, 64
H0W = D0N + W1 + W2
H1W = D1N + W3

_info = plsc.get_sparse_core_info()
_NC, _NS = _info.num_cores, _info.num_subcores
NW = _NC * _NS
BPW = B // NW
L = 16


@functools.partial(
    pl.kernel,
    mesh=plsc.VectorSubcoreMesh(core_axis_name="c", subcore_axis_name="s"),
    out_type=(
        jax.ShapeDtypeStruct((B, H0W), jnp.float32),
        jax.ShapeDtypeStruct((B, H1W), jnp.float32),
    ),
    scratch_types=[
        pltpu.VMEM((BPW,), jnp.int32),
        pltpu.VMEM((BPW,), jnp.int32),
        pltpu.VMEM((BPW,), jnp.int32),
        pltpu.VMEM((BPW, D0N), jnp.float32),
        pltpu.VMEM((BPW, D1N), jnp.float32),
        pltpu.VMEM((BPW, W1), jnp.float32),
        pltpu.VMEM((BPW, W2), jnp.float32),
        pltpu.VMEM((BPW, W3), jnp.float32),
        pltpu.VMEM((BPW, H0W), jnp.float32),
        pltpu.VMEM((BPW, H1W), jnp.float32),
        pltpu.SemaphoreType.DMA,
        pltpu.SemaphoreType.DMA,
    ],
    compiler_params=pltpu.CompilerParams(use_tc_tiling_on_sc=False, needs_layout_passes=False),
)
def _embed_concat(c0a, c0b, c1, x0n, x1n, e1, e2, e3, h0, h1,
                  i0a_v, i0b_v, i1_v, n0_v, n1_v, g0a_v, g0b_v, g1_v,
                  buf0, buf1, sem0, sem1):
    wid = lax.axis_index("s") * _NC + lax.axis_index("c")
    base = wid * BPW
    rows = pl.ds(base, BPW)
    pltpu.sync_copy(c0a.at[rows], i0a_v)
    pltpu.sync_copy(c0b.at[rows], i0b_v)
    pltpu.sync_copy(c1.at[rows], i1_v)
    cp_n0 = pltpu.async_copy(x0n.at[rows, :], n0_v, sem0)
    g0a = pltpu.async_copy(e1.at[i0a_v], g0a_v, sem0)
    g0b = pltpu.async_copy(e2.at[i0b_v], g0b_v, sem0)
    cp_n1 = pltpu.async_copy(x1n.at[rows, :], n1_v, sem1)
    g1 = pltpu.async_copy(e3.at[i1_v], g1_v, sem1)

    iota = lax.iota(jnp.int32, L)
    m10 = iota < D0N

    cp_n0.wait()
    g0a.wait()
    g0b.wait()

    def body0(r, _):
        rsp = jnp.full((L,), r, jnp.int32)
        x = plsc.load_gather(n0_v, [rsp, iota], mask=m10)
        plsc.store_scatter(buf0, [rsp, iota], x, mask=m10)
        for c in (0, 16):
            x = g0a_v[r, pl.ds(c, L)]
            plsc.store_scatter(buf0, [rsp, iota + (D0N + c)], x)
        x = g0b_v[r, pl.ds(0, L)]
        plsc.store_scatter(buf0, [rsp, iota + (D0N + W1)], x)
        return _

    lax.fori_loop(0, BPW, body0, None)
    out0 = pltpu.async_copy(buf0, h0.at[rows], sem0)

    cp_n1.wait()
    g1.wait()

    def body1(r, _):
        rsp = jnp.full((L,), r, jnp.int32)
        for c in (0, D1N - L):
            x = plsc.load_gather(n1_v, [rsp, iota + c])
            plsc.store_scatter(buf1, [rsp, iota + c], x)
        for c in (0, 16, 32, 48):
            x = g1_v[r, pl.ds(c, L)]
            plsc.store_scatter(buf1, [rsp, iota + (D1N + c)], x)
        return _

    lax.fori_loop(0, BPW, body1, None)
    out0.wait()
    pltpu.sync_copy(buf1, h1.at[rows])


def kernel(X0_num, X0_cat, X1_num, X1_cat, X2_num, E_cat1, E_cat2, E_cat3):
    c0a = X0_cat[:, 0]
    c0b = X0_cat[:, 1]
    c1 = X1_cat[:, 0]
    H0, H1 = _embed_concat(c0a, c0b, c1, X0_num, X1_num, E_cat1, E_cat2, E_cat3)
    return (H0, H1, X2_num)
